# baseline (speedup 1.0000x reference)
import functools

import jax
import jax.numpy as jnp
from jax import lax
from jax.experimental import pallas as pl
from jax.experimental.pallas import tpu as pltpu
from jax.experimental.pallas import tpu_sc as plsc

NN = 10000
NP = 10240
EE = 320000
GG = 64
NC = 2
NS = 16
NW = NC * NS
CB1 = 64
NCH1 = 160
CB2 = 128
NCH2 = 80
EPW = 10240
EP = EPW * NW
CW = 8
RPT = NP // NS


def _zero_rows_bf(buf, rows, width):
    def zrow(i, c):
        for k in range(width // 32):
            buf[i, pl.ds(k * 32, 32)] = jnp.zeros((32,), jnp.bfloat16)
        return c
    lax.fori_loop(0, rows, zrow, 0, unroll=2)


def _compute_relu_bf(u_rows, v_rows, b, cb, H):
    zero = jnp.zeros((32,), jnp.bfloat16)

    def row(i, c):
        for k in range(H // 32):
            sl = pl.ds(k * 32, 32)
            u_rows[b, i, sl] = jnp.maximum(u_rows[b, i, sl] + v_rows[b, i, sl],
                                           zero)
        return c
    lax.fori_loop(0, cb, row, 0, unroll=2)


def _edge_body_l1(u_hbm, v_hbm, dsti_hbm, srci_hbm, onez_hbm, acc_out,
                  cnt_out, dst_v, src_v, u_rows, v_rows, ones_v,
                  acc_sh, cnt_sh, *sems):
    H = 128
    su, sv, ss, sc = sems[0:2], sems[2:4], sems[4:6], sems[6:8]
    sdi, ssi = sems[8:12], sems[12:16]
    cid = lax.axis_index("c")
    sid = lax.axis_index("s")
    wid = cid * NS + sid

    def idx_issue(cc, i4):
        pltpu.async_copy(dsti_hbm.at[wid].at[cc], dst_v.at[i4], sdi[i4])
        pltpu.async_copy(srci_hbm.at[wid].at[cc], src_v.at[i4], ssi[i4])

    def idx_wait(i4):
        pltpu.make_async_copy(dsti_hbm.at[0].at[0], dst_v.at[i4],
                              sdi[i4]).wait()
        pltpu.make_async_copy(srci_hbm.at[0].at[0], src_v.at[i4],
                              ssi[i4]).wait()

    def gather_issue(b, i4):
        pltpu.async_copy(u_hbm.at[dst_v.at[i4]], u_rows.at[b], su[b])
        pltpu.async_copy(v_hbm.at[src_v.at[i4]], v_rows.at[b], sv[b])

    def gather_wait(b):
        pltpu.make_async_copy(u_hbm.at[dst_v.at[0]], u_rows.at[b],
                              su[b]).wait()
        pltpu.make_async_copy(v_hbm.at[src_v.at[0]], v_rows.at[b],
                              sv[b]).wait()

    def scatter_issue(b, i4):
        pltpu.async_copy(u_rows.at[b], acc_sh.at[dst_v.at[i4]], ss[b],
                         add=True)
        pltpu.async_copy(ones_v, cnt_sh.at[dst_v.at[i4]], sc[b], add=True)

    def scatter_wait(b):
        pltpu.make_async_copy(u_rows.at[b], acc_sh.at[dst_v.at[0]],
                              ss[b]).wait()
        pltpu.make_async_copy(ones_v, cnt_sh.at[dst_v.at[0]], sc[b]).wait()

    _zero_rows_bf(u_rows.at[0], CB1, H)
    for j in range(RPT // CB1):
        pltpu.sync_copy(u_rows.at[0],
                        acc_sh.at[pl.ds(sid * RPT + j * CB1, CB1)])
    pltpu.sync_copy(onez_hbm.at[0], ones_v)
    for j in range(RPT // CB1):
        pltpu.sync_copy(ones_v,
                        cnt_sh.at[pl.ds(sid * RPT + j * CB1, CB1)])
    pltpu.sync_copy(onez_hbm.at[1], ones_v)

    plsc.subcore_barrier()

    def step(cc, b, i4, i4n, i4n2, first=False, no_idx=False, no_next=False):
        gather_wait(b)
        _compute_relu_bf(u_rows, v_rows, b, CB1, H)
        scatter_issue(b, i4)
        if not no_next:
            idx_wait(i4n)
            if not first:
                scatter_wait(1 - b)
            gather_issue(1 - b, i4n)
            if not no_idx:
                idx_issue(cc + 2, i4n2)

    idx_issue(0, 0)
    idx_issue(1, 1)
    idx_wait(0)
    gather_issue(0, 0)
    step(0, 0, 0, 1, 2, first=True)
    step(1, 1, 1, 2, 3)

    @pl.loop(2, NCH1 - 2, step=4)
    def _(base):
        for j in range(4):
            step(base + j, j % 2, (2 + j) % 4, (3 + j) % 4, j % 4)

    step(NCH1 - 2, 0, 2, 3, 0, no_idx=True)
    step(NCH1 - 1, 1, 3, 0, 0, no_next=True)
    scatter_wait(0)
    scatter_wait(1)

    plsc.subcore_barrier()
    pltpu.sync_copy(acc_sh.at[pl.ds(sid * RPT, RPT)],
                    acc_out.at[cid].at[pl.ds(sid * RPT, RPT)])
    pltpu.sync_copy(cnt_sh.at[pl.ds(sid * RPT, RPT)],
                    cnt_out.at[cid].at[pl.ds(sid * RPT, RPT)])


def _edge_body_hn(u_hbm, v_hbm, dsti_hbm, srci_hbm, acc_out,
                  dst_s, src_s, u_rows, v_rows, acc_sh, *sems, H):
    su, sv, ss = sems[0:2], sems[2:4], sems[4:6]
    cid = lax.axis_index("c")
    sid = lax.axis_index("s")
    wid = cid * NS + sid

    pltpu.sync_copy(dsti_hbm.at[wid], dst_s)
    pltpu.sync_copy(srci_hbm.at[wid], src_s)

    def gather_issue(b, cc):
        pltpu.async_copy(u_hbm.at[dst_s.at[cc]], u_rows.at[b], su[b])
        pltpu.async_copy(v_hbm.at[src_s.at[cc]], v_rows.at[b], sv[b])

    def gather_wait(b):
        pltpu.make_async_copy(u_hbm.at[dst_s.at[0]], u_rows.at[b],
                              su[b]).wait()
        pltpu.make_async_copy(v_hbm.at[src_s.at[0]], v_rows.at[b],
                              sv[b]).wait()

    def scatter_issue(b, cc):
        pltpu.async_copy(u_rows.at[b], acc_sh.at[dst_s.at[cc]], ss[b],
                         add=True)

    def scatter_wait(b):
        pltpu.make_async_copy(u_rows.at[b], acc_sh.at[dst_s.at[0]],
                              ss[b]).wait()

    _zero_rows_bf(u_rows.at[0], CB2, H)
    for j in range(RPT // CB2):
        pltpu.sync_copy(u_rows.at[0],
                        acc_sh.at[pl.ds(sid * RPT + j * CB2, CB2)])
    plsc.subcore_barrier()

    def step(cc, b, first=False, no_next=False):
        gather_wait(b)
        _compute_relu_bf(u_rows, v_rows, b, CB2, H)
        scatter_issue(b, cc)
        if not no_next:
            if not first:
                scatter_wait(1 - b)
            gather_issue(1 - b, cc + 1)

    gather_issue(0, 0)
    step(0, 0, first=True)
    step(1, 1)

    @pl.loop(2, NCH2 - 2, step=2)
    def _(base):
        for j in range(2):
            step(base + j, j)

    step(NCH2 - 2, 0)
    step(NCH2 - 1, 1, no_next=True)
    scatter_wait(0)
    scatter_wait(1)

    plsc.subcore_barrier()
    pltpu.sync_copy(acc_sh.at[pl.ds(sid * RPT, RPT)],
                    acc_out.at[cid].at[pl.ds(sid * RPT, RPT)])


def _make_edge_kernel(H, with_cnt):
    mesh = plsc.VectorSubcoreMesh(core_axis_name="c", subcore_axis_name="s",
                                  num_cores=NC, num_subcores=NS)
    acc_t = jax.ShapeDtypeStruct((NC, NP, H), jnp.bfloat16)
    if with_cnt:
        out_type = (acc_t, jax.ShapeDtypeStruct((NC, NP, CW), jnp.float32))
        scratch = [
            pltpu.VMEM((4, CB1), jnp.int32),
            pltpu.VMEM((4, CB1), jnp.int32),
            pltpu.VMEM((2, CB1, H), jnp.bfloat16),
            pltpu.VMEM((2, CB1, H), jnp.bfloat16),
            pltpu.VMEM((CB1, CW), jnp.float32),
            pltpu.VMEM_SHARED((NP, H), jnp.bfloat16),
            pltpu.VMEM_SHARED((NP, CW), jnp.float32),
        ] + [pltpu.SemaphoreType.DMA] * 16
        body = _edge_body_l1
    else:
        out_type = acc_t
        scratch = [
            pltpu.VMEM((NCH2, CB2), jnp.int32),
            pltpu.VMEM((NCH2, CB2), jnp.int32),
            pltpu.VMEM((2, CB2, H), jnp.bfloat16),
            pltpu.VMEM((2, CB2, H), jnp.bfloat16),
            pltpu.VMEM_SHARED((NP, H), jnp.bfloat16),
        ] + [pltpu.SemaphoreType.DMA] * 6
        body = functools.partial(_edge_body_hn, H=H)
    return pl.kernel(body, out_type=out_type,
                     mesh=mesh, scratch_types=tuple(scratch),
                     compiler_params=pltpu.CompilerParams(
                         use_tc_tiling_on_sc=False,
                         needs_layout_passes=False))


def _tc0_body(x_ref, new_ref, neb_ref, w1_ref, b1_ref, u_ref, v_ref):
    h = jnp.dot(x_ref[...], new_ref[...],
                preferred_element_type=jnp.float32) + neb_ref[...]
    F = h.shape[1]
    wl = w1_ref[:F, :]
    wr = w1_ref[F:, :]
    u_ref[...] = (jnp.dot(h, wl - wr, preferred_element_type=jnp.float32)
                  + b1_ref[...]).astype(jnp.bfloat16)
    v_ref[...] = jnp.dot(h, wr,
                         preferred_element_type=jnp.float32).astype(jnp.bfloat16)


def _tc_mid_body(acc_ref, cnt_ref, w2_ref, b2_ref, bng_ref, bnb_ref,
                 w1n_ref, b1n_ref, u_ref, v_ref):
    cnt = cnt_ref[0, :, 0:1] + cnt_ref[1, :, 0:1]
    mean = (acc_ref[0].astype(jnp.float32) + acc_ref[1].astype(jnp.float32)
            ) / jnp.maximum(cnt, 1.0)
    g = jnp.dot(mean, w2_ref[...],
                preferred_element_type=jnp.float32) + b2_ref[...]
    g = jnp.where(cnt > 0.0, g, 0.0)
    rm = (lax.broadcasted_iota(jnp.int32, (NP, 1), 0) < NN).astype(jnp.float32)
    mu = jnp.sum(g * rm, axis=0, keepdims=True) / NN
    d = (g - mu) * rm
    var = jnp.sum(d * d, axis=0, keepdims=True) / NN
    h = jnp.maximum((g - mu) / jnp.sqrt(var + 1e-5) * bng_ref[...] + bnb_ref[...], 0.0)
    h = h * rm
    F = h.shape[1]
    wl = w1n_ref[:F, :]
    wr = w1n_ref[F:, :]
    u_ref[...] = (jnp.dot(h, wl - wr, preferred_element_type=jnp.float32)
                  + b1n_ref[...]).astype(jnp.bfloat16)
    v_ref[...] = jnp.dot(h, wr,
                         preferred_element_type=jnp.float32).astype(jnp.bfloat16)


def _tc_fin_body(acc_ref, cnt_ref, w2_ref, b2_ref, bng_ref, bnb_ref,
                 batch_ref, fcw_ref, fcb_ref, out_ref):
    cnt = cnt_ref[0, :, 0:1] + cnt_ref[1, :, 0:1]
    mean = (acc_ref[0].astype(jnp.float32) + acc_ref[1].astype(jnp.float32)
            ) / jnp.maximum(cnt, 1.0)
    g = jnp.dot(mean, w2_ref[...],
                preferred_element_type=jnp.float32) + b2_ref[...]
    g = jnp.where(cnt > 0.0, g, 0.0)
    rm = (lax.broadcasted_iota(jnp.int32, (NP, 1), 0) < NN).astype(jnp.float32)
    mu = jnp.sum(g * rm, axis=0, keepdims=True) / NN
    d = (g - mu) * rm
    var = jnp.sum(d * d, axis=0, keepdims=True) / NN
    h = jnp.maximum((g - mu) / jnp.sqrt(var + 1e-5) * bng_ref[...] + bnb_ref[...], 0.0)
    h = h * rm
    oh = (batch_ref[...] == lax.broadcasted_iota(jnp.int32, (GG, NP), 0)
          ).astype(jnp.float32)
    gs = jnp.dot(oh, h, preferred_element_type=jnp.float32)
    gc = jnp.sum(oh, axis=1, keepdims=True)
    pooled = gs / jnp.maximum(gc, 1.0)
    o = jnp.dot(pooled, fcw_ref[...],
                preferred_element_type=jnp.float32) + fcb_ref[...]
    out_ref[...] = jax.nn.sigmoid(o)


def _tc0(x_pad, ne_w, ne_b, m1_w1, m1_b1):
    return pl.pallas_call(
        _tc0_body,
        out_shape=(jax.ShapeDtypeStruct((NP, 128), jnp.bfloat16),
                   jax.ShapeDtypeStruct((NP, 128), jnp.bfloat16)),
    )(x_pad, ne_w, ne_b, m1_w1, m1_b1)


def _tc_mid(acc, cnt, w2, b2, bng, bnb, w1n, b1n, hn):
    return pl.pallas_call(
        _tc_mid_body,
        out_shape=(jax.ShapeDtypeStruct((NP, hn), jnp.bfloat16),
                   jax.ShapeDtypeStruct((NP, hn), jnp.bfloat16)),
    )(acc, cnt, w2, b2, bng, bnb, w1n, b1n)


def _tc_fin(acc, cnt, w2, b2, bng, bnb, batch_row, fc_w, fc_b):
    return pl.pallas_call(
        _tc_fin_body,
        out_shape=jax.ShapeDtypeStruct((GG, 5), jnp.float32),
    )(acc, cnt, w2, b2, bng, bnb, batch_row, fc_w, fc_b)


_edge_k1 = _make_edge_kernel(128, with_cnt=True)
_edge_k64 = _make_edge_kernel(64, with_cnt=False)
_edge_k32 = _make_edge_kernel(32, with_cnt=False)


@jax.jit
def kernel(x, edge_index, edge_attr, batch, ee_w, ee_b, ne_w, ne_b,
           m1_w1, m1_b1, m1_w2, m1_b2, m2_w1, m2_b1, m2_w2, m2_b2,
           m3_w1, m3_b1, m3_w2, m3_b2, bn1_g, bn1_b, bn2_g, bn2_b,
           bn3_g, bn3_b, fc_w, fc_b):
    src = edge_index[0]
    dst = edge_index[1]
    pad = NN + jnp.arange(EP - EE, dtype=jnp.int32) % (NP - NN)
    srcp = jnp.concatenate([src, pad])
    dstp = jnp.concatenate([dst, pad])
    srci1 = srcp.reshape(NW, NCH1, CB1)
    dsti1 = dstp.reshape(NW, NCH1, CB1)
    srci2 = srcp.reshape(NW, NCH2, CB2)
    dsti2 = dstp.reshape(NW, NCH2, CB2)
    onez = jnp.stack([jnp.zeros((CB1, CW), jnp.float32),
                      jnp.ones((CB1, CW), jnp.float32)])
    x_pad = jnp.pad(x, ((0, NP - NN), (0, 0)))
    batch_row = jnp.pad(batch, (0, NP - NN), constant_values=GG).reshape(1, NP)

    r1 = lambda a: a.reshape(1, -1)


    u1, v1 = _tc0(x_pad, ne_w, r1(ne_b), m1_w1, r1(m1_b1))
    acc1, cnt = _edge_k1(u1, v1, dsti1, srci1, onez)
    u2, v2 = _tc_mid(acc1, cnt, m1_w2, r1(m1_b2), r1(bn1_g), r1(bn1_b),
                     m2_w1, r1(m2_b1), 64)
    acc2 = _edge_k64(u2, v2, dsti2, srci2)
    u3, v3 = _tc_mid(acc2, cnt, m2_w2, r1(m2_b2), r1(bn2_g), r1(bn2_b),
                     m3_w1, r1(m3_b1), 32)
    acc3 = _edge_k32(u3, v3, dsti2, srci2)
    return _tc_fin(acc3, cnt, m3_w2, r1(m3_b2), r1(bn3_g), r1(bn3_b),
                   batch_row, fc_w, r1(fc_b))
